# baseline (speedup 1.0000x reference)
import functools

import jax
import jax.numpy as jnp
from jax import lax
from jax.experimental import pallas as pl
from jax.experimental.pallas import tpu as pltpu
from jax.experimental.pallas import tpu_sc as plsc

_NC = 2
_NS = 16
_L = 16


def _make_spmm(N, E, C):
    per_sub = E // _NS
    assert E % _NS == 0 and per_sub % C == 0
    n_chunks = per_sub // C
    ROWS_BIG = ((N + _NS - 1) // _NS + 7) // 8 * 8
    ROWS_LAST = N - ROWS_BIG * (_NS - 1)
    assert ROWS_LAST > 0 and ROWS_LAST % 8 == 0

    mesh = plsc.VectorSubcoreMesh(core_axis_name="c", subcore_axis_name="s")

    assert n_chunks >= 3
    NB = 3

    @functools.partial(
        pl.kernel,
        mesh=mesh,
        out_type=jax.ShapeDtypeStruct((_NC, N, 128), jnp.float32),
        scratch_types=(
            [pltpu.VMEM((1, C), jnp.int32) for _ in range(NB)]
            + [pltpu.VMEM((1, C), jnp.float32) for _ in range(NB)]
            + [pltpu.VMEM((1, C), jnp.int32) for _ in range(NB)]
            + [pltpu.VMEM((C, 128), jnp.float32) for _ in range(NB)]
            + [pltpu.VMEM_SHARED((N, 128), jnp.float32)]
            + [pltpu.SemaphoreType.DMA] * (5 * NB)
        ),
    )
    def spmm(rows_hbm, cols_hbm, vals_hbm, ego_hbm, zeros_hbm, out_hbm,
             *scr):
        colcs = scr[0:NB]
        valcs = scr[NB:2 * NB]
        rowcs = scr[2 * NB:3 * NB]
        bufs = scr[3 * NB:4 * NB]
        accum = scr[4 * NB]
        csems = scr[4 * NB + 1:4 * NB + 1 + NB]
        gsems = scr[4 * NB + 1 + NB:4 * NB + 1 + 2 * NB]
        vsems = scr[4 * NB + 1 + 2 * NB:4 * NB + 1 + 3 * NB]
        rsems = scr[4 * NB + 1 + 3 * NB:4 * NB + 1 + 4 * NB]
        ssems = scr[4 * NB + 1 + 4 * NB:4 * NB + 1 + 5 * NB]
        c = lax.axis_index("c")
        s = lax.axis_index("s")
        ego_c = ego_hbm.at[c]

        @pl.when(s < _NS - 1)
        def _():
            pltpu.sync_copy(zeros_hbm, accum.at[pl.ds(s * ROWS_BIG, ROWS_BIG)])

        @pl.when(s == _NS - 1)
        def _():
            pltpu.sync_copy(zeros_hbm.at[pl.ds(0, ROWS_LAST)],
                            accum.at[pl.ds((_NS - 1) * ROWS_BIG, ROWS_LAST)])

        plsc.subcore_barrier()

        def _when(pred, fn):
            if isinstance(pred, bool):
                if pred:
                    fn()
            else:
                pl.when(pred)(fn)

        def fetch_cols(j, b):
            pltpu.make_async_copy(cols_hbm.at[s, j], colcs[b], csems[b]).start()

        def wait_cols(j, b):
            pltpu.make_async_copy(cols_hbm.at[s, j], colcs[b], csems[b]).wait()

        def start_set(j, b):
            pltpu.make_async_copy(
                ego_c.at[colcs[b].at[0]], bufs[b], gsems[b]).start()
            pltpu.make_async_copy(vals_hbm.at[s, j], valcs[b], vsems[b]).start()
            pltpu.make_async_copy(rows_hbm.at[s, j], rowcs[b], rsems[b]).start()

        def wait_set(j, b):
            pltpu.make_async_copy(
                ego_c.at[colcs[b].at[0]], bufs[b], gsems[b]).wait()
            pltpu.make_async_copy(vals_hbm.at[s, j], valcs[b], vsems[b]).wait()

        def wait_rows(j, b):
            pltpu.make_async_copy(rows_hbm.at[s, j], rowcs[b], rsems[b]).wait()

        def start_scatter(b):
            pltpu.make_async_copy(
                bufs[b], accum.at[rowcs[b].at[0]], ssems[b]).start(add=True)

        def wait_scatter(b):
            pltpu.make_async_copy(
                bufs[b], accum.at[rowcs[b].at[0]], ssems[b]).wait()

        fetch_cols(0, 0)
        fetch_cols(1, 1)
        wait_cols(0, 0)
        start_set(0, 0)

        def compute(j, b):
            buf = bufs[b]
            valc = valcs[b]

            for g16 in range(C // _L):
                v16 = valc[0, pl.ds(g16 * _L, _L)]
                r0 = g16 * _L
                for r16 in range(_L):
                    vv = lax.gather(
                        v16, jnp.full((_L, 1), r16, jnp.int32),
                        lax.GatherDimensionNumbers(
                            offset_dims=(), collapsed_slice_dims=(0,),
                            start_index_map=(0,)),
                        slice_sizes=(1,),
                        mode=lax.GatherScatterMode.PROMISE_IN_BOUNDS)
                    r = r0 + r16
                    for k in range(128 // _L):
                        buf[r, pl.ds(k * _L, _L)] = (
                            buf[r, pl.ds(k * _L, _L)] * vv)

        def process_chunk(j, b):
            b1 = (b + 1) % NB
            b2 = (b + 2) % NB
            wait_set(j, b)

            def _issue_next():
                _when(j >= 2, lambda: wait_scatter(b1))
                wait_cols(j + 1, b1)
                start_set(j + 1, b1)

            _when(j + 1 < n_chunks, _issue_next)
            compute(j, b)
            _when(j + 2 < n_chunks, lambda: fetch_cols(j + 2, b2))
            wait_rows(j, b)
            start_scatter(b)

        def triple_body(h, carry):
            for t in range(NB):
                j = NB * h + t
                process_chunk(j, t)
            return carry

        n_triples = n_chunks // NB
        lax.fori_loop(0, n_triples, triple_body, 0)
        for j in range(n_triples * NB, n_chunks):
            process_chunk(j, j % NB)
        for b in range(NB):
            wait_scatter(b)
        plsc.subcore_barrier()

        @pl.when(s < _NS - 1)
        def _():
            pltpu.sync_copy(accum.at[pl.ds(s * ROWS_BIG, ROWS_BIG)],
                            out_hbm.at[c, pl.ds(s * ROWS_BIG, ROWS_BIG)])

        @pl.when(s == _NS - 1)
        def _():
            pltpu.sync_copy(accum.at[pl.ds((_NS - 1) * ROWS_BIG, ROWS_LAST)],
                            out_hbm.at[c, pl.ds((_NS - 1) * ROWS_BIG, ROWS_LAST)])

    return spmm


def _attention(x, Wq, Wk, Wv, Wf, bf):
    F, D = x.shape
    scale = float(1.0 / (float(D) ** 0.5))

    def body(x_ref, wq_ref, wk_ref, wv_ref, wf_ref, bf_ref, out_ref):
        xv = x_ref[...]
        q = lax.dot(xv, wq_ref[...], precision=lax.Precision.HIGHEST)
        k = lax.dot(xv, wk_ref[...], precision=lax.Precision.HIGHEST)
        v = lax.dot(xv, wv_ref[...], precision=lax.Precision.HIGHEST)
        s = lax.dot_general(q, k, (((1,), (1,)), ((), ())),
                            precision=lax.Precision.HIGHEST) * scale
        m = jnp.max(s, axis=-1, keepdims=True)
        p = jnp.exp(s - m)
        p = p / jnp.sum(p, axis=-1, keepdims=True)
        av = lax.dot(p, v, precision=lax.Precision.HIGHEST)
        out_ref[...] = (lax.dot(av, wf_ref[...], precision=lax.Precision.HIGHEST)
                        + bf_ref[...])

    return pl.pallas_call(
        body,
        out_shape=jax.ShapeDtypeStruct((F, Wf.shape[1]), jnp.float32),
    )(x, Wq, Wk, Wv, Wf, bf.reshape(1, -1))


def _leaky(x):
    return jnp.where(x >= 0, x, x * 0.01)


def _layer_math(side_ref, e, wg_ref, bg_ref, wb_ref, bb_ref):
    side = jnp.concatenate([side_ref[0], side_ref[1]], axis=-1)
    sum_emb = _leaky(lax.dot(side, wg_ref[...]) + bg_ref[...])
    bi = _leaky(lax.dot(e * side, wb_ref[...]) + bb_ref[...])
    out = sum_emb + bi
    nrm = jnp.sqrt(jnp.sum(out * out, axis=-1, keepdims=True))
    return out, out / jnp.maximum(nrm, 1e-12)


def _dense_layer1(ego, side2, Wg, bg, Wb, bb, BN=1000):
    N, D = ego.shape
    assert N % BN == 0

    def body(side_ref, ego_ref, wg_ref, bg_ref, wb_ref, bb_ref,
             ego2_out_ref, ego_out_ref, norm_out_ref):
        out, normed = _layer_math(side_ref, ego_ref[...],
                                  wg_ref, bg_ref, wb_ref, bb_ref)
        ego2_out_ref[0] = out[:, :128]
        ego2_out_ref[1] = out[:, 128:]
        ego_out_ref[...] = out
        norm_out_ref[...] = normed

    grid = (N // BN,)
    return pl.pallas_call(
        body,
        grid=grid,
        in_specs=[
            pl.BlockSpec((_NC, BN, 128), lambda i: (0, i, 0)),
            pl.BlockSpec((BN, D), lambda i: (i, 0)),
            pl.BlockSpec((D, D), lambda i: (0, 0)),
            pl.BlockSpec((1, D), lambda i: (0, 0)),
            pl.BlockSpec((D, D), lambda i: (0, 0)),
            pl.BlockSpec((1, D), lambda i: (0, 0)),
        ],
        out_specs=[
            pl.BlockSpec((_NC, BN, 128), lambda i: (0, i, 0)),
            pl.BlockSpec((BN, D), lambda i: (i, 0)),
            pl.BlockSpec((BN, D), lambda i: (i, 0)),
        ],
        out_shape=[
            jax.ShapeDtypeStruct((_NC, N, 128), jnp.float32),
            jax.ShapeDtypeStruct((N, D), jnp.float32),
            jax.ShapeDtypeStruct((N, D), jnp.float32),
        ],
    )(side2, ego, Wg, bg.reshape(1, -1), Wb, bb.reshape(1, -1))


def _dense_out(ego0, n1, ego1, side2, Wg, bg, Wb, bb, row_off, rows, BN):
    N, D = ego1.shape
    assert rows % BN == 0 and row_off % BN == 0
    off = row_off // BN

    def body(side_ref, ego1_ref, ego0_ref, n1_ref,
             wg_ref, bg_ref, wb_ref, bb_ref, out_ref):
        _, normed = _layer_math(side_ref, ego1_ref[...],
                                wg_ref, bg_ref, wb_ref, bb_ref)
        out_ref[...] = jnp.concatenate(
            [ego0_ref[...], n1_ref[...], normed], axis=-1)

    grid = (rows // BN,)
    return pl.pallas_call(
        body,
        grid=grid,
        in_specs=[
            pl.BlockSpec((_NC, BN, 128), lambda i: (0, i + off, 0)),
            pl.BlockSpec((BN, D), lambda i: (i + off, 0)),
            pl.BlockSpec((BN, D), lambda i: (i + off, 0)),
            pl.BlockSpec((BN, D), lambda i: (i + off, 0)),
            pl.BlockSpec((D, D), lambda i: (0, 0)),
            pl.BlockSpec((1, D), lambda i: (0, 0)),
            pl.BlockSpec((D, D), lambda i: (0, 0)),
            pl.BlockSpec((1, D), lambda i: (0, 0)),
        ],
        out_specs=pl.BlockSpec((BN, 3 * D), lambda i: (i, 0)),
        out_shape=jax.ShapeDtypeStruct((rows, 3 * D), jnp.float32),
    )(side2, ego1, ego0, n1, Wg, bg.reshape(1, -1), Wb, bb.reshape(1, -1))


def kernel(edge_index, adj_values, features_data, user_emb, item_emb,
           Wq, Wk, Wv, Wf, bf, Wg0, bg0, Wg1, bg1, Wb0, bb0, Wb1, bb1):
    n_users = user_emb.shape[0]
    n_items = item_emb.shape[0]
    n_feat = features_data.shape[0]
    N = n_users + n_items + n_feat
    D = user_emb.shape[1]
    E = adj_values.shape[0]
    C = 80
    per_sub = E // _NS
    n_chunks = per_sub // C

    atten_score = _attention(features_data, Wq, Wk, Wv, Wf, bf)
    ego0 = jnp.concatenate([user_emb, item_emb, atten_score], axis=0)
    ego0_2 = jnp.stack([ego0[:, :128], ego0[:, 128:]], axis=0)

    row = edge_index[0].astype(jnp.int32)
    col = edge_index[1].astype(jnp.int32)
    rows3 = row.reshape(_NS, n_chunks, 1, C)
    cols2 = col.reshape(_NS, n_chunks, 1, C)
    vals3 = adj_values.reshape(_NS, n_chunks, 1, C)
    rows_big = ((N + _NS - 1) // _NS + 7) // 8 * 8
    zeros = jnp.zeros((rows_big, 128), jnp.float32)

    spmm = _make_spmm(N, E, C)

    side2_1 = spmm(rows3, cols2, vals3, ego0_2, zeros)
    ego1_2, ego1, n1 = _dense_layer1(ego0, side2_1, Wg0, bg0, Wb0, bb0)
    side2_2 = spmm(rows3, cols2, vals3, ego1_2, zeros)
    u_g = _dense_out(ego0, n1, ego1, side2_2, Wg1, bg1, Wb1, bb1,
                     0, n_users, 1000)
    i_g = _dense_out(ego0, n1, ego1, side2_2, Wg1, bg1, Wb1, bb1,
                     n_users, n_items, 200)
    return (u_g, i_g, atten_score)


# speedup vs baseline: 1.1990x; 1.1990x over previous
import functools

import jax
import jax.numpy as jnp
from jax import lax
from jax.experimental import pallas as pl
from jax.experimental.pallas import tpu as pltpu
from jax.experimental.pallas import tpu_sc as plsc

_NC = 2
_NS = 16
_L = 16


def _make_spmm(N, E, C):
    per_sub = E // _NS
    assert E % _NS == 0 and per_sub % C == 0
    n_chunks = per_sub // C
    ROWS_BIG = ((N + _NS - 1) // _NS + 7) // 8 * 8
    ROWS_LAST = N - ROWS_BIG * (_NS - 1)
    assert ROWS_LAST > 0 and ROWS_LAST % 8 == 0

    mesh = plsc.VectorSubcoreMesh(core_axis_name="c", subcore_axis_name="s")

    assert n_chunks >= 3
    NB = 3

    @functools.partial(
        pl.kernel,
        mesh=mesh,
        out_type=jax.ShapeDtypeStruct((_NC, N, 128), jnp.float32),
        scratch_types=(
            [pltpu.VMEM((1, C), jnp.int32) for _ in range(NB)]
            + [pltpu.VMEM((1, C), jnp.float32) for _ in range(NB)]
            + [pltpu.VMEM((1, C), jnp.int32) for _ in range(NB)]
            + [pltpu.VMEM((C, 128), jnp.float32) for _ in range(NB)]
            + [pltpu.VMEM_SHARED((N, 128), jnp.float32)]
            + [pltpu.SemaphoreType.DMA] * (5 * NB)
        ),
    )
    def spmm(rows_hbm, cols_hbm, vals_hbm, ego_hbm, zeros_hbm, out_hbm,
             *scr):
        colcs = scr[0:NB]
        valcs = scr[NB:2 * NB]
        rowcs = scr[2 * NB:3 * NB]
        bufs = scr[3 * NB:4 * NB]
        accum = scr[4 * NB]
        csems = scr[4 * NB + 1:4 * NB + 1 + NB]
        gsems = scr[4 * NB + 1 + NB:4 * NB + 1 + 2 * NB]
        vsems = scr[4 * NB + 1 + 2 * NB:4 * NB + 1 + 3 * NB]
        rsems = scr[4 * NB + 1 + 3 * NB:4 * NB + 1 + 4 * NB]
        ssems = scr[4 * NB + 1 + 4 * NB:4 * NB + 1 + 5 * NB]
        c = lax.axis_index("c")
        s = lax.axis_index("s")
        ego_c = ego_hbm.at[c]

        @pl.when(s < _NS - 1)
        def _():
            pltpu.sync_copy(zeros_hbm, accum.at[pl.ds(s * ROWS_BIG, ROWS_BIG)])

        @pl.when(s == _NS - 1)
        def _():
            pltpu.sync_copy(zeros_hbm.at[pl.ds(0, ROWS_LAST)],
                            accum.at[pl.ds((_NS - 1) * ROWS_BIG, ROWS_LAST)])

        plsc.subcore_barrier()

        def _when(pred, fn):
            if isinstance(pred, bool):
                if pred:
                    fn()
            else:
                pl.when(pred)(fn)

        def fetch_cols(j, b):
            pltpu.make_async_copy(cols_hbm.at[s, j], colcs[b], csems[b]).start()

        def wait_cols(j, b):
            pltpu.make_async_copy(cols_hbm.at[s, j], colcs[b], csems[b]).wait()

        def start_set(j, b):
            pltpu.make_async_copy(
                ego_c.at[colcs[b].at[0]], bufs[b], gsems[b]).start()
            pltpu.make_async_copy(vals_hbm.at[s, j], valcs[b], vsems[b]).start()
            pltpu.make_async_copy(rows_hbm.at[s, j], rowcs[b], rsems[b]).start()

        def wait_set(j, b):
            pltpu.make_async_copy(
                ego_c.at[colcs[b].at[0]], bufs[b], gsems[b]).wait()
            pltpu.make_async_copy(vals_hbm.at[s, j], valcs[b], vsems[b]).wait()

        def wait_rows(j, b):
            pltpu.make_async_copy(rows_hbm.at[s, j], rowcs[b], rsems[b]).wait()

        def start_scatter(b):
            pltpu.make_async_copy(
                bufs[b], accum.at[rowcs[b].at[0]], ssems[b]).start(add=True)

        def wait_scatter(b):
            pltpu.make_async_copy(
                bufs[b], accum.at[rowcs[b].at[0]], ssems[b]).wait()

        fetch_cols(0, 0)
        fetch_cols(1, 1)
        wait_cols(0, 0)
        start_set(0, 0)

        def compute(j, b):
            buf = bufs[b]
            valc = valcs[b]

            def group_body(g16, carry2):
                v16 = valc[0, pl.ds(g16 * _L, _L)]
                r0 = g16 * _L
                for r16 in range(_L):
                    vv = lax.gather(
                        v16, jnp.full((_L, 1), r16, jnp.int32),
                        lax.GatherDimensionNumbers(
                            offset_dims=(), collapsed_slice_dims=(0,),
                            start_index_map=(0,)),
                        slice_sizes=(1,),
                        mode=lax.GatherScatterMode.PROMISE_IN_BOUNDS)
                    r = r0 + r16
                    for k in range(128 // _L):
                        buf[r, pl.ds(k * _L, _L)] = (
                            buf[r, pl.ds(k * _L, _L)] * vv)
                return carry2

            lax.fori_loop(0, C // _L, group_body, 0)

        def process_chunk(j, b):
            b1 = (b + 1) % NB
            b2 = (b + 2) % NB
            wait_set(j, b)

            def _issue_next():
                _when(j >= 2, lambda: wait_scatter(b1))
                wait_cols(j + 1, b1)
                start_set(j + 1, b1)

            _when(j + 1 < n_chunks, _issue_next)
            compute(j, b)
            _when(j + 2 < n_chunks, lambda: fetch_cols(j + 2, b2))
            wait_rows(j, b)
            start_scatter(b)

        def triple_body(h, carry):
            for t in range(NB):
                j = NB * h + t
                process_chunk(j, t)
            return carry

        n_triples = n_chunks // NB
        lax.fori_loop(0, n_triples, triple_body, 0)
        for j in range(n_triples * NB, n_chunks):
            process_chunk(j, j % NB)
        for b in range(NB):
            wait_scatter(b)
        plsc.subcore_barrier()

        @pl.when(s < _NS - 1)
        def _():
            pltpu.sync_copy(accum.at[pl.ds(s * ROWS_BIG, ROWS_BIG)],
                            out_hbm.at[c, pl.ds(s * ROWS_BIG, ROWS_BIG)])

        @pl.when(s == _NS - 1)
        def _():
            pltpu.sync_copy(accum.at[pl.ds((_NS - 1) * ROWS_BIG, ROWS_LAST)],
                            out_hbm.at[c, pl.ds((_NS - 1) * ROWS_BIG, ROWS_LAST)])

    return spmm


def _attention(x, Wq, Wk, Wv, Wf, bf):
    F, D = x.shape
    scale = float(1.0 / (float(D) ** 0.5))

    def body(x_ref, wq_ref, wk_ref, wv_ref, wf_ref, bf_ref, out_ref):
        xv = x_ref[...]
        q = lax.dot(xv, wq_ref[...], precision=lax.Precision.HIGHEST)
        k = lax.dot(xv, wk_ref[...], precision=lax.Precision.HIGHEST)
        v = lax.dot(xv, wv_ref[...], precision=lax.Precision.HIGHEST)
        s = lax.dot_general(q, k, (((1,), (1,)), ((), ())),
                            precision=lax.Precision.HIGHEST) * scale
        m = jnp.max(s, axis=-1, keepdims=True)
        p = jnp.exp(s - m)
        p = p / jnp.sum(p, axis=-1, keepdims=True)
        av = lax.dot(p, v, precision=lax.Precision.HIGHEST)
        out_ref[...] = (lax.dot(av, wf_ref[...], precision=lax.Precision.HIGHEST)
                        + bf_ref[...])

    return pl.pallas_call(
        body,
        out_shape=jax.ShapeDtypeStruct((F, Wf.shape[1]), jnp.float32),
    )(x, Wq, Wk, Wv, Wf, bf.reshape(1, -1))


def _leaky(x):
    return jnp.where(x >= 0, x, x * 0.01)


def _layer_math(side_ref, e, wg_ref, bg_ref, wb_ref, bb_ref):
    side = jnp.concatenate([side_ref[0], side_ref[1]], axis=-1)
    sum_emb = _leaky(lax.dot(side, wg_ref[...]) + bg_ref[...])
    bi = _leaky(lax.dot(e * side, wb_ref[...]) + bb_ref[...])
    out = sum_emb + bi
    nrm = jnp.sqrt(jnp.sum(out * out, axis=-1, keepdims=True))
    return out, out / jnp.maximum(nrm, 1e-12)


def _dense_layer1(ego, side2, Wg, bg, Wb, bb, BN=1000):
    N, D = ego.shape
    assert N % BN == 0

    def body(side_ref, ego_ref, wg_ref, bg_ref, wb_ref, bb_ref,
             ego2_out_ref, ego_out_ref, norm_out_ref):
        out, normed = _layer_math(side_ref, ego_ref[...],
                                  wg_ref, bg_ref, wb_ref, bb_ref)
        ego2_out_ref[0] = out[:, :128]
        ego2_out_ref[1] = out[:, 128:]
        ego_out_ref[...] = out
        norm_out_ref[...] = normed

    grid = (N // BN,)
    return pl.pallas_call(
        body,
        grid=grid,
        in_specs=[
            pl.BlockSpec((_NC, BN, 128), lambda i: (0, i, 0)),
            pl.BlockSpec((BN, D), lambda i: (i, 0)),
            pl.BlockSpec((D, D), lambda i: (0, 0)),
            pl.BlockSpec((1, D), lambda i: (0, 0)),
            pl.BlockSpec((D, D), lambda i: (0, 0)),
            pl.BlockSpec((1, D), lambda i: (0, 0)),
        ],
        out_specs=[
            pl.BlockSpec((_NC, BN, 128), lambda i: (0, i, 0)),
            pl.BlockSpec((BN, D), lambda i: (i, 0)),
            pl.BlockSpec((BN, D), lambda i: (i, 0)),
        ],
        out_shape=[
            jax.ShapeDtypeStruct((_NC, N, 128), jnp.float32),
            jax.ShapeDtypeStruct((N, D), jnp.float32),
            jax.ShapeDtypeStruct((N, D), jnp.float32),
        ],
    )(side2, ego, Wg, bg.reshape(1, -1), Wb, bb.reshape(1, -1))


def _dense_out(ego0, n1, ego1, side2, Wg, bg, Wb, bb, row_off, rows, BN):
    N, D = ego1.shape
    assert rows % BN == 0 and row_off % BN == 0
    off = row_off // BN

    def body(side_ref, ego1_ref, ego0_ref, n1_ref,
             wg_ref, bg_ref, wb_ref, bb_ref, out_ref):
        _, normed = _layer_math(side_ref, ego1_ref[...],
                                wg_ref, bg_ref, wb_ref, bb_ref)
        out_ref[...] = jnp.concatenate(
            [ego0_ref[...], n1_ref[...], normed], axis=-1)

    grid = (rows // BN,)
    return pl.pallas_call(
        body,
        grid=grid,
        in_specs=[
            pl.BlockSpec((_NC, BN, 128), lambda i: (0, i + off, 0)),
            pl.BlockSpec((BN, D), lambda i: (i + off, 0)),
            pl.BlockSpec((BN, D), lambda i: (i + off, 0)),
            pl.BlockSpec((BN, D), lambda i: (i + off, 0)),
            pl.BlockSpec((D, D), lambda i: (0, 0)),
            pl.BlockSpec((1, D), lambda i: (0, 0)),
            pl.BlockSpec((D, D), lambda i: (0, 0)),
            pl.BlockSpec((1, D), lambda i: (0, 0)),
        ],
        out_specs=pl.BlockSpec((BN, 3 * D), lambda i: (i, 0)),
        out_shape=jax.ShapeDtypeStruct((rows, 3 * D), jnp.float32),
    )(side2, ego1, ego0, n1, Wg, bg.reshape(1, -1), Wb, bb.reshape(1, -1))


def kernel(edge_index, adj_values, features_data, user_emb, item_emb,
           Wq, Wk, Wv, Wf, bf, Wg0, bg0, Wg1, bg1, Wb0, bb0, Wb1, bb1):
    n_users = user_emb.shape[0]
    n_items = item_emb.shape[0]
    n_feat = features_data.shape[0]
    N = n_users + n_items + n_feat
    D = user_emb.shape[1]
    E = adj_values.shape[0]
    C = 80
    per_sub = E // _NS
    n_chunks = per_sub // C

    atten_score = _attention(features_data, Wq, Wk, Wv, Wf, bf)
    ego0 = jnp.concatenate([user_emb, item_emb, atten_score], axis=0)
    ego0_2 = jnp.stack([ego0[:, :128], ego0[:, 128:]], axis=0)

    row = edge_index[0].astype(jnp.int32)
    col = edge_index[1].astype(jnp.int32)
    rows3 = row.reshape(_NS, n_chunks, 1, C)
    cols2 = col.reshape(_NS, n_chunks, 1, C)
    vals3 = adj_values.reshape(_NS, n_chunks, 1, C)
    rows_big = ((N + _NS - 1) // _NS + 7) // 8 * 8
    zeros = jnp.zeros((rows_big, 128), jnp.float32)

    spmm = _make_spmm(N, E, C)

    side2_1 = spmm(rows3, cols2, vals3, ego0_2, zeros)
    ego1_2, ego1, n1 = _dense_layer1(ego0, side2_1, Wg0, bg0, Wb0, bb0)
    side2_2 = spmm(rows3, cols2, vals3, ego1_2, zeros)
    u_g = _dense_out(ego0, n1, ego1, side2_2, Wg1, bg1, Wb1, bb1,
                     0, n_users, 1000)
    i_g = _dense_out(ego0, n1, ego1, side2_2, Wg1, bg1, Wb1, bb1,
                     n_users, n_items, 200)
    return (u_g, i_g, atten_score)
